# baseline (speedup 1.0000x reference)
import functools

import jax
import jax.numpy as jnp
from jax import lax
from jax.experimental import pallas as pl
from jax.experimental.pallas import tpu as pltpu
from jax.experimental.pallas import tpu_sc as plsc

N = 10000
D = 128
M = 320000
NC, NS = 2, 16
NW = NC * NS
CHUNK = 64
NP = 10240
ROWS_PER_SUB = NP // NS
MP = 327680
PER_SUB = MP // NW
NCHUNK = PER_SUB // CHUNK
MROWS = MP // CHUNK
CCHUNK = 128
CNCHUNK = PER_SUB // CCHUNK
CMROWS = MP // CCHUNK
TRASH = 10200

_f32 = jnp.float32


def _mesh():
    return plsc.VectorSubcoreMesh(
        core_axis_name="c", subcore_axis_name="s", num_cores=NC, num_subcores=NS
    )


CG = 128
NG = PER_SUB // CG
GMROWS = MP // CG


@functools.partial(
    pl.kernel,
    out_type=jax.ShapeDtypeStruct((MP, D), _f32),
    mesh=_mesh(),
    scratch_types=[
        pltpu.VMEM((4, CG), jnp.int32),
        [pltpu.VMEM((CG, D), _f32)] * 2,
        [pltpu.SemaphoreType.DMA] * 4,
        [pltpu.SemaphoreType.DMA] * 2,
        [pltpu.SemaphoreType.DMA] * 2,
        pltpu.VMEM_SHARED((NP, D), _f32),
    ],
)
def _sc_permute(tbl, gidx, out, gvc, rows, semi, semg, semw, stbl):
    c = lax.axis_index("c")
    s = lax.axis_index("s")
    r0 = s * ROWS_PER_SUB
    pltpu.sync_copy(tbl.at[pl.ds(r0, ROWS_PER_SUB)], stbl.at[pl.ds(r0, ROWS_PER_SUB)])
    w = c * NS + s
    row0 = w * NG
    base = w * PER_SUB

    def idx_load(j, ib):
        pltpu.async_copy(gidx.at[row0 + j], gvc.at[ib], semi[ib])

    def idx_wait(j, ib):
        pltpu.make_async_copy(gidx.at[row0 + j], gvc.at[ib], semi[ib]).wait()

    def gather_start(ib, b):
        pltpu.async_copy(stbl.at[gvc.at[ib]], rows[b], semg[b])

    def gather_wait(ib, b):
        pltpu.make_async_copy(stbl.at[gvc.at[ib]], rows[b], semg[b]).wait()

    def write_start(j, b):
        pltpu.async_copy(rows[b], out.at[pl.ds(base + j * CG, CG)], semw[b])

    def write_wait(j, b):
        pltpu.make_async_copy(rows[b], out.at[pl.ds(base + j * CG, CG)], semw[b]).wait()

    for j in range(4):
        idx_load(j, j)
    plsc.subcore_barrier()
    idx_wait(0, 0)
    gather_start(0, 0)

    def body(t, carry):
        for u in range(4):
            j = t * 4 + u
            b = u % 2
            gather_wait(u, b)
            write_start(j, b)

            @pl.when(j >= 1)
            def _():
                write_wait(j - 1, 1 - b)

            @pl.when(j + 1 < NG)
            def _():
                idx_wait(j + 1, (u + 1) % 4)
                gather_start((u + 1) % 4, 1 - b)

            @pl.when(j + 4 < NG)
            def _():
                idx_load(j + 4, u)

        return carry

    lax.fori_loop(0, NG // 4, body, 0)
    write_wait(NG - 1, (NG - 1) % 2)


NBUF = 4
NIB = 8
LR = 2
LI = 6
NITER = NCHUNK // NIB


@functools.partial(
    pl.kernel,
    out_type=jax.ShapeDtypeStruct((NC, NP, D), _f32),
    mesh=_mesh(),
    scratch_types=[
        pltpu.VMEM((NIB, CHUNK), jnp.int32),
        [pltpu.VMEM((CHUNK, D), _f32)] * NBUF,
        [pltpu.SemaphoreType.DMA] * NIB,
        [pltpu.SemaphoreType.DMA] * NBUF,
        [pltpu.SemaphoreType.DMA] * NBUF,
        pltpu.VMEM_SHARED((NP, D), _f32),
    ],
)
def _sc_scat(perm, sidx, zeros, out, svc, rows, semi, semr, sems, acc):
    c = lax.axis_index("c")
    s = lax.axis_index("s")
    r0 = s * ROWS_PER_SUB
    pltpu.sync_copy(zeros, acc.at[pl.ds(r0, ROWS_PER_SUB)])
    w = c * NS + s
    row0 = w * NCHUNK
    base = w * PER_SUB

    def idx_load(j, ib):
        pltpu.async_copy(sidx.at[row0 + j], svc.at[ib], semi[ib])

    def idx_wait(j, ib):
        pltpu.make_async_copy(sidx.at[row0 + j], svc.at[ib], semi[ib]).wait()

    def read_start(j, b):
        pltpu.async_copy(perm.at[pl.ds(base + j * CHUNK, CHUNK)], rows[b], semr[b])

    def read_wait(j, b):
        pltpu.make_async_copy(
            perm.at[pl.ds(base + j * CHUNK, CHUNK)], rows[b], semr[b]
        ).wait()

    def scat_start(ib, b):
        pltpu.async_copy(rows[b], acc.at[svc.at[ib]], sems[b], add=True)

    def scat_wait(ib, b):
        pltpu.make_async_copy(rows[b], acc.at[svc.at[ib]], sems[b]).wait()

    for j in range(LI):
        idx_load(j, j)
    plsc.subcore_barrier()
    for j in range(LR):
        read_start(j, j)

    def body(t, carry):
        for u in range(NIB):
            j = t * NIB + u
            b = u % NBUF
            read_wait(j, b)
            idx_wait(j, u)
            scat_start(u, b)

            @pl.when(j + LR < NCHUNK)
            def _():
                @pl.when(j >= NBUF - LR)
                def _():
                    scat_wait((u - NBUF + LR) % NIB, (u + LR) % NBUF)

                read_start(j + LR, (u + LR) % NBUF)

            @pl.when(j + LI < NCHUNK)
            def _():
                idx_load(j + LI, (u + LI) % NIB)

        return carry

    lax.fori_loop(0, NITER, body, 0)
    for k in range(NCHUNK - NBUF, NCHUNK):
        scat_wait(k % NIB, k % NBUF)
    plsc.subcore_barrier()
    pltpu.sync_copy(
        acc.at[pl.ds(r0, ROWS_PER_SUB)], out.at[c, pl.ds(r0, ROWS_PER_SUB)]
    )


def _sc_pass(tbl, gidx128, sidx64, zeros):
    return _sc_scat(_sc_permute(tbl, gidx128), sidx64, zeros)


@functools.partial(
    pl.kernel,
    out_type=(
        jax.ShapeDtypeStruct((NC, NP, D), _f32),
        jax.ShapeDtypeStruct((NC, NP, D), _f32),
    ),
    mesh=_mesh(),
    scratch_types=[
        pltpu.VMEM((CNCHUNK, CCHUNK), jnp.int32),
        pltpu.VMEM((CNCHUNK, CCHUNK), jnp.int32),
        pltpu.VMEM((CCHUNK, D), _f32),
        [pltpu.SemaphoreType.DMA] * 4,
        pltpu.VMEM_SHARED((NP, D), _f32),
    ],
)
def _sc_counts(nidx, eidx, ones, zeros, outn, oute, nv, ev, onesv, sem, acc):
    c = lax.axis_index("c")
    s = lax.axis_index("s")
    r0 = s * ROWS_PER_SUB
    pltpu.sync_copy(ones, onesv)
    w = c * NS + s
    pltpu.sync_copy(nidx.at[pl.ds(w * CNCHUNK, CNCHUNK)], nv)
    pltpu.sync_copy(eidx.at[pl.ds(w * CNCHUNK, CNCHUNK)], ev)
    for (iv, o) in ((nv, outn), (ev, oute)):
        pltpu.sync_copy(zeros, acc.at[pl.ds(r0, ROWS_PER_SUB)])
        plsc.subcore_barrier()

        def body(t, carry, iv=iv):
            for u in range(4):
                j = t * 4 + u

                @pl.when(t >= 1)
                def _():
                    pltpu.make_async_copy(
                        onesv, acc.at[iv.at[j - 4]], sem[u]
                    ).wait()

                pltpu.async_copy(onesv, acc.at[iv.at[j]], sem[u], add=True)
            return carry

        lax.fori_loop(0, CNCHUNK // 4, body, 0)
        for k in range(CNCHUNK - 4, CNCHUNK):
            pltpu.make_async_copy(onesv, acc.at[iv.at[k]], sem[k % 4]).wait()
        plsc.subcore_barrier()
        pltpu.sync_copy(
            acc.at[pl.ds(r0, ROWS_PER_SUB)],
            o.at[c, pl.ds(r0, ROWS_PER_SUB)],
        )
        plsc.subcore_barrier()


_BLK = 1000
_BLKP = 640


def _mm_body(x_ref, w_ref, o_ref):
    o_ref[...] = jnp.dot(x_ref[...], w_ref[...], preferred_element_type=_f32)


_tc_mm = pl.pallas_call(
    _mm_body,
    grid=(NP // _BLKP,),
    in_specs=[
        pl.BlockSpec((_BLKP, D), lambda i: (i, 0)),
        pl.BlockSpec((D, D), lambda i: (0, 0)),
    ],
    out_specs=pl.BlockSpec((_BLKP, D), lambda i: (i, 0)),
    out_shape=jax.ShapeDtypeStruct((NP, D), _f32),
)


def _scaleinv(c0, c1):
    cnt = c0[:, 0:1] + c1[:, 0:1]
    return jnp.where(cnt > 0, 1.0 / cnt, 0.0)


def _comb_a_body(p0_ref, p1_ref, c0_ref, c1_ref, o_ref):
    o_ref[...] = _scaleinv(c0_ref[...], c1_ref[...]) * (p0_ref[...] + p1_ref[...])


_tc_comb_a = pl.pallas_call(
    _comb_a_body,
    grid=(NP // _BLKP,),
    in_specs=[
        pl.BlockSpec((_BLKP, D), lambda i: (i, 0)),
        pl.BlockSpec((_BLKP, D), lambda i: (i, 0)),
        pl.BlockSpec((_BLKP, 16), lambda i: (i, 0)),
        pl.BlockSpec((_BLKP, 16), lambda i: (i, 0)),
    ],
    out_specs=pl.BlockSpec((_BLKP, D), lambda i: (i, 0)),
    out_shape=jax.ShapeDtypeStruct((NP, D), _f32),
)


def _comb_b_mm_body(q0_ref, q1_ref, c0_ref, c1_ref, b_ref, w_ref, o_ref):
    h = _scaleinv(c0_ref[...], c1_ref[...]) * (q0_ref[...] + q1_ref[...]) + b_ref[...]
    h = jnp.where(h >= 0, h, 0.01 * h)
    o_ref[...] = jnp.dot(h, w_ref[...], preferred_element_type=_f32)


_tc_comb_b_mm = pl.pallas_call(
    _comb_b_mm_body,
    grid=(NP // _BLKP,),
    in_specs=[
        pl.BlockSpec((_BLKP, D), lambda i: (i, 0)),
        pl.BlockSpec((_BLKP, D), lambda i: (i, 0)),
        pl.BlockSpec((_BLKP, 16), lambda i: (i, 0)),
        pl.BlockSpec((_BLKP, 16), lambda i: (i, 0)),
        pl.BlockSpec((1, D), lambda i: (0, 0)),
        pl.BlockSpec((D, D), lambda i: (0, 0)),
    ],
    out_specs=pl.BlockSpec((_BLKP, D), lambda i: (i, 0)),
    out_shape=jax.ShapeDtypeStruct((NP, D), _f32),
)


def _final_body(q0_ref, q1_ref, c0_ref, c1_ref, b_ref, o_ref):
    h = _scaleinv(c0_ref[...], c1_ref[...]) * (q0_ref[...] + q1_ref[...]) + b_ref[...]
    g = lax.broadcasted_iota(jnp.int32, (8, D), 0)
    r = lax.broadcasted_iota(jnp.int32, (8, D), 1) // 16
    sel = (g == r).astype(_f32)
    o_ref[...] = jnp.dot(sel, h, preferred_element_type=_f32)


_tc_final = pl.pallas_call(
    _final_body,
    out_shape=jax.ShapeDtypeStruct((8, D), _f32),
)


def kernel(x, edge_index, W0, b0, W1, b1, W2, b2):
    nidx = edge_index[0].astype(jnp.int32)
    eidx = edge_index[1].astype(jnp.int32)
    pad_s = jnp.full((MP - M,), TRASH, jnp.int32)
    pad_g = jnp.zeros((MP - M,), jnp.int32)
    nflat_s = jnp.concatenate([nidx, pad_s])
    eflat_s = jnp.concatenate([eidx, pad_s])
    nidx_s = nflat_s.reshape(MROWS, CHUNK)
    eidx_s = eflat_s.reshape(MROWS, CHUNK)
    nidx_g = jnp.concatenate([nidx, pad_g]).reshape(GMROWS, CG)
    eidx_g = jnp.concatenate([eidx, pad_g]).reshape(GMROWS, CG)
    zeros = jnp.zeros((ROWS_PER_SUB, D), _f32)
    ones = jnp.ones((CCHUNK, D), _f32)
    x_pad = jnp.concatenate([x, jnp.zeros((NP - N, D), _f32)])

    cn, ce = _sc_counts(
        nflat_s.reshape(CMROWS, CCHUNK), eflat_s.reshape(CMROWS, CCHUNK),
        ones, zeros,
    )
    cn0, cn1 = cn[0, :, 0:16], cn[1, :, 0:16]
    ce0, ce1 = ce[0, :, 0:16], ce[1, :, 0:16]
    b0r, b1r, b2r = b0.reshape(1, D), b1.reshape(1, D), b2.reshape(1, D)

    xt = _tc_mm(x_pad, W0)
    p = _sc_pass(xt, nidx_g, eidx_s, zeros)
    hef = _tc_comb_a(p[0], p[1], ce0, ce1)
    q = _sc_pass(hef, eidx_g, nidx_s, zeros)
    xt = _tc_comb_b_mm(q[0], q[1], cn0, cn1, b0r, W1)

    p = _sc_pass(xt, nidx_g, eidx_s, zeros)
    hef = _tc_comb_a(p[0], p[1], ce0, ce1)
    q = _sc_pass(hef, eidx_g, nidx_s, zeros)
    xt = _tc_comb_b_mm(q[0], q[1], cn0, cn1, b1r, W2)

    p = _sc_pass(xt, nidx_g, eidx_s, zeros)
    hef = _tc_comb_a(p[0], p[1], ce0, ce1)
    q = _sc_pass(hef, eidx_g, nidx_s, zeros)
    return _tc_final(q[0][0:128], q[1][0:128], cn0[0:128], cn1[0:128], b2r)


# speedup vs baseline: 1.0659x; 1.0659x over previous
import functools

import jax
import jax.numpy as jnp
from jax import lax
from jax.experimental import pallas as pl
from jax.experimental.pallas import tpu as pltpu
from jax.experimental.pallas import tpu_sc as plsc

N = 10000
D = 128
M = 320000
NC, NS = 2, 16
NW = NC * NS
CHUNK = 64
NP = 10240
ROWS_PER_SUB = NP // NS
MP = 327680
PER_SUB = MP // NW
NCHUNK = PER_SUB // CHUNK
MROWS = MP // CHUNK
CCHUNK = 128
CNCHUNK = PER_SUB // CCHUNK
CMROWS = MP // CCHUNK
TRASH = 10200

_f32 = jnp.float32


def _mesh():
    return plsc.VectorSubcoreMesh(
        core_axis_name="c", subcore_axis_name="s", num_cores=NC, num_subcores=NS
    )


CG = 128
NG = PER_SUB // CG
GMROWS = MP // CG


@functools.partial(
    pl.kernel,
    out_type=jax.ShapeDtypeStruct((MP, D), _f32),
    mesh=_mesh(),
    scratch_types=[
        pltpu.VMEM((4, CG), jnp.int32),
        [pltpu.VMEM((CG, D), _f32)] * 2,
        [pltpu.SemaphoreType.DMA] * 4,
        [pltpu.SemaphoreType.DMA] * 2,
        [pltpu.SemaphoreType.DMA] * 2,
        pltpu.VMEM_SHARED((NP, D), _f32),
    ],
)
def _sc_permute(tbl, gidx, out, gvc, rows, semi, semg, semw, stbl):
    c = lax.axis_index("c")
    s = lax.axis_index("s")
    r0 = s * ROWS_PER_SUB
    pltpu.sync_copy(tbl.at[pl.ds(r0, ROWS_PER_SUB)], stbl.at[pl.ds(r0, ROWS_PER_SUB)])
    w = c * NS + s
    row0 = w * NG
    base = w * PER_SUB

    def idx_load(j, ib):
        pltpu.async_copy(gidx.at[row0 + j], gvc.at[ib], semi[ib])

    def idx_wait(j, ib):
        pltpu.make_async_copy(gidx.at[row0 + j], gvc.at[ib], semi[ib]).wait()

    def gather_start(ib, b):
        pltpu.async_copy(stbl.at[gvc.at[ib]], rows[b], semg[b])

    def gather_wait(ib, b):
        pltpu.make_async_copy(stbl.at[gvc.at[ib]], rows[b], semg[b]).wait()

    def write_start(j, b):
        pltpu.async_copy(rows[b], out.at[pl.ds(base + j * CG, CG)], semw[b])

    def write_wait(j, b):
        pltpu.make_async_copy(rows[b], out.at[pl.ds(base + j * CG, CG)], semw[b]).wait()

    for j in range(4):
        idx_load(j, j)
    plsc.subcore_barrier()
    idx_wait(0, 0)
    gather_start(0, 0)

    def body(t, carry):
        for u in range(4):
            j = t * 4 + u
            b = u % 2
            gather_wait(u, b)
            write_start(j, b)

            @pl.when(j >= 1)
            def _():
                write_wait(j - 1, 1 - b)

            @pl.when(j + 1 < NG)
            def _():
                idx_wait(j + 1, (u + 1) % 4)
                gather_start((u + 1) % 4, 1 - b)

            @pl.when(j + 4 < NG)
            def _():
                idx_load(j + 4, u)

        return carry

    lax.fori_loop(0, NG // 4, body, 0)
    write_wait(NG - 1, (NG - 1) % 2)


NBUF = 4
NIB = 8
LR = 3
LI = 6
NITER = NCHUNK // NIB


@functools.partial(
    pl.kernel,
    out_type=jax.ShapeDtypeStruct((NC, NP, D), _f32),
    mesh=_mesh(),
    scratch_types=[
        pltpu.VMEM((NIB, CHUNK), jnp.int32),
        [pltpu.VMEM((CHUNK, D), _f32)] * NBUF,
        [pltpu.SemaphoreType.DMA] * NIB,
        [pltpu.SemaphoreType.DMA] * NBUF,
        [pltpu.SemaphoreType.DMA] * NBUF,
        pltpu.VMEM_SHARED((NP, D), _f32),
    ],
)
def _sc_scat(perm, sidx, zeros, out, svc, rows, semi, semr, sems, acc):
    c = lax.axis_index("c")
    s = lax.axis_index("s")
    r0 = s * ROWS_PER_SUB
    pltpu.sync_copy(zeros, acc.at[pl.ds(r0, ROWS_PER_SUB)])
    w = c * NS + s
    row0 = w * NCHUNK
    base = w * PER_SUB

    def idx_load(j, ib):
        pltpu.async_copy(sidx.at[row0 + j], svc.at[ib], semi[ib])

    def idx_wait(j, ib):
        pltpu.make_async_copy(sidx.at[row0 + j], svc.at[ib], semi[ib]).wait()

    def read_start(j, b):
        pltpu.async_copy(perm.at[pl.ds(base + j * CHUNK, CHUNK)], rows[b], semr[b])

    def read_wait(j, b):
        pltpu.make_async_copy(
            perm.at[pl.ds(base + j * CHUNK, CHUNK)], rows[b], semr[b]
        ).wait()

    def scat_start(ib, b):
        pltpu.async_copy(rows[b], acc.at[svc.at[ib]], sems[b], add=True)

    def scat_wait(ib, b):
        pltpu.make_async_copy(rows[b], acc.at[svc.at[ib]], sems[b]).wait()

    for j in range(LI):
        idx_load(j, j)
    plsc.subcore_barrier()
    for j in range(LR):
        read_start(j, j)

    def body(t, carry):
        for u in range(NIB):
            j = t * NIB + u
            b = u % NBUF
            read_wait(j, b)
            idx_wait(j, u)
            scat_start(u, b)

            @pl.when(j + LR < NCHUNK)
            def _():
                @pl.when(j >= NBUF - LR)
                def _():
                    scat_wait((u - NBUF + LR) % NIB, (u + LR) % NBUF)

                read_start(j + LR, (u + LR) % NBUF)

            @pl.when(j + LI < NCHUNK)
            def _():
                idx_load(j + LI, (u + LI) % NIB)

        return carry

    lax.fori_loop(0, NITER, body, 0)
    for k in range(NCHUNK - NBUF, NCHUNK):
        scat_wait(k % NIB, k % NBUF)
    plsc.subcore_barrier()
    pltpu.sync_copy(
        acc.at[pl.ds(r0, ROWS_PER_SUB)], out.at[c, pl.ds(r0, ROWS_PER_SUB)]
    )


def _sc_pass(tbl, gidx128, sidx64, zeros):
    return _sc_scat(_sc_permute(tbl, gidx128), sidx64, zeros)


@functools.partial(
    pl.kernel,
    out_type=(
        jax.ShapeDtypeStruct((NC, NP, D), _f32),
        jax.ShapeDtypeStruct((NC, NP, D), _f32),
    ),
    mesh=_mesh(),
    scratch_types=[
        pltpu.VMEM((CNCHUNK, CCHUNK), jnp.int32),
        pltpu.VMEM((CNCHUNK, CCHUNK), jnp.int32),
        pltpu.VMEM((CCHUNK, D), _f32),
        [pltpu.SemaphoreType.DMA] * 4,
        pltpu.VMEM_SHARED((NP, D), _f32),
    ],
)
def _sc_counts(nidx, eidx, ones, zeros, outn, oute, nv, ev, onesv, sem, acc):
    c = lax.axis_index("c")
    s = lax.axis_index("s")
    r0 = s * ROWS_PER_SUB
    pltpu.sync_copy(ones, onesv)
    w = c * NS + s
    pltpu.sync_copy(nidx.at[pl.ds(w * CNCHUNK, CNCHUNK)], nv)
    pltpu.sync_copy(eidx.at[pl.ds(w * CNCHUNK, CNCHUNK)], ev)
    for (iv, o) in ((nv, outn), (ev, oute)):
        pltpu.sync_copy(zeros, acc.at[pl.ds(r0, ROWS_PER_SUB)])
        plsc.subcore_barrier()

        def body(t, carry, iv=iv):
            for u in range(4):
                j = t * 4 + u

                @pl.when(t >= 1)
                def _():
                    pltpu.make_async_copy(
                        onesv, acc.at[iv.at[j - 4]], sem[u]
                    ).wait()

                pltpu.async_copy(onesv, acc.at[iv.at[j]], sem[u], add=True)
            return carry

        lax.fori_loop(0, CNCHUNK // 4, body, 0)
        for k in range(CNCHUNK - 4, CNCHUNK):
            pltpu.make_async_copy(onesv, acc.at[iv.at[k]], sem[k % 4]).wait()
        plsc.subcore_barrier()
        pltpu.sync_copy(
            acc.at[pl.ds(r0, ROWS_PER_SUB)],
            o.at[c, pl.ds(r0, ROWS_PER_SUB)],
        )
        plsc.subcore_barrier()


_BLK = 1000
_BLKP = 640


def _mm_body(x_ref, w_ref, o_ref):
    o_ref[...] = jnp.dot(x_ref[...], w_ref[...], preferred_element_type=_f32)


_tc_mm = pl.pallas_call(
    _mm_body,
    grid=(NP // _BLKP,),
    in_specs=[
        pl.BlockSpec((_BLKP, D), lambda i: (i, 0)),
        pl.BlockSpec((D, D), lambda i: (0, 0)),
    ],
    out_specs=pl.BlockSpec((_BLKP, D), lambda i: (i, 0)),
    out_shape=jax.ShapeDtypeStruct((NP, D), _f32),
)


def _scaleinv(c0, c1):
    cnt = c0[:, 0:1] + c1[:, 0:1]
    return jnp.where(cnt > 0, 1.0 / cnt, 0.0)


def _comb_a_body(p0_ref, p1_ref, c0_ref, c1_ref, o_ref):
    o_ref[...] = _scaleinv(c0_ref[...], c1_ref[...]) * (p0_ref[...] + p1_ref[...])


_tc_comb_a = pl.pallas_call(
    _comb_a_body,
    grid=(NP // _BLKP,),
    in_specs=[
        pl.BlockSpec((_BLKP, D), lambda i: (i, 0)),
        pl.BlockSpec((_BLKP, D), lambda i: (i, 0)),
        pl.BlockSpec((_BLKP, 16), lambda i: (i, 0)),
        pl.BlockSpec((_BLKP, 16), lambda i: (i, 0)),
    ],
    out_specs=pl.BlockSpec((_BLKP, D), lambda i: (i, 0)),
    out_shape=jax.ShapeDtypeStruct((NP, D), _f32),
)


def _comb_b_mm_body(q0_ref, q1_ref, c0_ref, c1_ref, b_ref, w_ref, o_ref):
    h = _scaleinv(c0_ref[...], c1_ref[...]) * (q0_ref[...] + q1_ref[...]) + b_ref[...]
    h = jnp.where(h >= 0, h, 0.01 * h)
    o_ref[...] = jnp.dot(h, w_ref[...], preferred_element_type=_f32)


_tc_comb_b_mm = pl.pallas_call(
    _comb_b_mm_body,
    grid=(NP // _BLKP,),
    in_specs=[
        pl.BlockSpec((_BLKP, D), lambda i: (i, 0)),
        pl.BlockSpec((_BLKP, D), lambda i: (i, 0)),
        pl.BlockSpec((_BLKP, 16), lambda i: (i, 0)),
        pl.BlockSpec((_BLKP, 16), lambda i: (i, 0)),
        pl.BlockSpec((1, D), lambda i: (0, 0)),
        pl.BlockSpec((D, D), lambda i: (0, 0)),
    ],
    out_specs=pl.BlockSpec((_BLKP, D), lambda i: (i, 0)),
    out_shape=jax.ShapeDtypeStruct((NP, D), _f32),
)


def _final_body(q0_ref, q1_ref, c0_ref, c1_ref, b_ref, o_ref):
    h = _scaleinv(c0_ref[...], c1_ref[...]) * (q0_ref[...] + q1_ref[...]) + b_ref[...]
    g = lax.broadcasted_iota(jnp.int32, (8, D), 0)
    r = lax.broadcasted_iota(jnp.int32, (8, D), 1) // 16
    sel = (g == r).astype(_f32)
    o_ref[...] = jnp.dot(sel, h, preferred_element_type=_f32)


_tc_final = pl.pallas_call(
    _final_body,
    out_shape=jax.ShapeDtypeStruct((8, D), _f32),
)


def kernel(x, edge_index, W0, b0, W1, b1, W2, b2):
    nidx = edge_index[0].astype(jnp.int32)
    eidx = edge_index[1].astype(jnp.int32)
    pad_s = jnp.full((MP - M,), TRASH, jnp.int32)
    pad_g = jnp.zeros((MP - M,), jnp.int32)
    nflat_s = jnp.concatenate([nidx, pad_s])
    eflat_s = jnp.concatenate([eidx, pad_s])
    nidx_s = nflat_s.reshape(MROWS, CHUNK)
    eidx_s = eflat_s.reshape(MROWS, CHUNK)
    nidx_g = jnp.concatenate([nidx, pad_g]).reshape(GMROWS, CG)
    eidx_g = jnp.concatenate([eidx, pad_g]).reshape(GMROWS, CG)
    zeros = jnp.zeros((ROWS_PER_SUB, D), _f32)
    ones = jnp.ones((CCHUNK, D), _f32)
    x_pad = jnp.concatenate([x, jnp.zeros((NP - N, D), _f32)])

    cn, ce = _sc_counts(
        nflat_s.reshape(CMROWS, CCHUNK), eflat_s.reshape(CMROWS, CCHUNK),
        ones, zeros,
    )
    cn0, cn1 = cn[0, :, 0:16], cn[1, :, 0:16]
    ce0, ce1 = ce[0, :, 0:16], ce[1, :, 0:16]
    b0r, b1r, b2r = b0.reshape(1, D), b1.reshape(1, D), b2.reshape(1, D)

    xt = _tc_mm(x_pad, W0)
    p = _sc_pass(xt, nidx_g, eidx_s, zeros)
    hef = _tc_comb_a(p[0], p[1], ce0, ce1)
    q = _sc_pass(hef, eidx_g, nidx_s, zeros)
    xt = _tc_comb_b_mm(q[0], q[1], cn0, cn1, b0r, W1)

    p = _sc_pass(xt, nidx_g, eidx_s, zeros)
    hef = _tc_comb_a(p[0], p[1], ce0, ce1)
    q = _sc_pass(hef, eidx_g, nidx_s, zeros)
    xt = _tc_comb_b_mm(q[0], q[1], cn0, cn1, b1r, W2)

    p = _sc_pass(xt, nidx_g, eidx_s, zeros)
    hef = _tc_comb_a(p[0], p[1], ce0, ce1)
    q = _sc_pass(hef, eidx_g, nidx_s, zeros)
    return _tc_final(q[0][0:128], q[1][0:128], cn0[0:128], cn1[0:128], b2r)
